# Initial kernel scaffold; baseline (speedup 1.0000x reference)
import jax
import jax.numpy as jnp
from jax.experimental import pallas as pl


def kernel(inputs, inputs_ema, targets, features):
    raise NotImplementedError("write your pallas kernel here")


# speedup vs baseline: 1.7873x; 1.7873x over previous
import functools

import jax
import jax.numpy as jnp
from jax.experimental import pallas as pl
from jax.experimental.pallas import tpu as pltpu

B, D, N = 1024, 1024, 8192
TEMP = 0.05
TN = 2048
NT = N // TN


def _fused_loss_kernel(x_ref, f_ref, t_ref, out_ref, inv_ref, s_ref, ta_ref):
    j = pl.program_id(0)

    @pl.when(j == 0)
    def _init():
        xf = x_ref[...]
        norm = jnp.sqrt(jnp.sum(xf * xf, axis=1, keepdims=True))
        inv_ref[...] = 1.0 / (jnp.maximum(norm, 1e-12) * TEMP)
        s_ref[...] = jnp.zeros_like(s_ref)
        ta_ref[...] = jnp.zeros_like(ta_ref)

    xb = x_ref[...].astype(jnp.bfloat16)
    logits = jax.lax.dot_general(
        xb, f_ref[...], (((1,), (1,)), ((), ())),
        preferred_element_type=jnp.float32,
    ) * inv_ref[...]
    s_ref[...] += jnp.sum(jnp.exp(logits), axis=1, keepdims=True)
    col = jax.lax.broadcasted_iota(jnp.int32, (B, TN), 1) + j * TN
    ta_ref[...] += jnp.sum(
        jnp.where(col == t_ref[...], logits, 0.0), axis=1, keepdims=True
    )

    @pl.when(j == NT - 1)
    def _fin():
        out_ref[0, 0] = jnp.sum(jnp.log(s_ref[...]) - ta_ref[...]) / B


@jax.jit
def _fused_loss(inputs, targets, features_bf16):
    out = pl.pallas_call(
        _fused_loss_kernel,
        grid=(NT,),
        in_specs=[
            pl.BlockSpec((B, D), lambda j: (0, 0)),
            pl.BlockSpec((TN, D), lambda j: (j, 0)),
            pl.BlockSpec((B, 1), lambda j: (0, 0)),
        ],
        out_specs=pl.BlockSpec(memory_space=pltpu.SMEM),
        out_shape=jax.ShapeDtypeStruct((1, 1), jnp.float32),
        scratch_shapes=[
            pltpu.VMEM((B, 1), jnp.float32),
            pltpu.VMEM((B, 1), jnp.float32),
            pltpu.VMEM((B, 1), jnp.float32),
        ],
        compiler_params=pltpu.CompilerParams(
            dimension_semantics=("arbitrary",),
        ),
    )(inputs, features_bf16, targets)
    return out[0, 0]


def kernel(inputs, inputs_ema, targets, features):
    del inputs_ema
    tgt = targets.astype(jnp.int32).reshape(B, 1)
    return _fused_loss(inputs, tgt, features.astype(jnp.bfloat16))
